# baseline (speedup 1.0000x reference)
import math

import jax
import jax.numpy as jnp
from jax.experimental import pallas as pl
from jax.experimental.pallas import tpu as pltpu

_IN = 512
_OUT = 512
_SCALE = 1.0 / math.sqrt(_IN)
_CHUNK = 2048
_NBUF = 3


def _body(x_hbm, mu_ref, sig_ref, gate_ref, bias_ref, scores_hbm, out_hbm,
          xbuf, sbuf, obuf, in_sems, ssems, osems):
    n_chunks = x_hbm.shape[0] // _CHUNK
    mu = mu_ref[:]
    keys16 = (mu * jax.nn.softplus(sig_ref[:])).astype(jnp.bfloat16)
    mu16 = mu.astype(jnp.bfloat16)
    gate = gate_ref[:]
    bias = bias_ref[:]

    def in_copy(c, slot):
        return pltpu.make_async_copy(
            x_hbm.at[pl.ds(c * _CHUNK, _CHUNK), :], xbuf.at[slot],
            in_sems.at[slot])

    def s_copy(c, slot):
        return pltpu.make_async_copy(
            sbuf.at[slot], scores_hbm.at[pl.ds(c * _CHUNK, _CHUNK), :],
            ssems.at[slot])

    def o_copy(c, slot):
        return pltpu.make_async_copy(
            obuf.at[slot], out_hbm.at[pl.ds(c * _CHUNK, _CHUNK), :],
            osems.at[slot])

    for s in range(_NBUF):
        in_copy(s, s).start()

    def step(c, carry):
        slot = jax.lax.rem(c, _NBUF)
        in_copy(c, slot).wait()

        @pl.when(c >= _NBUF)
        def _():
            s_copy(c, slot).wait()
            o_copy(c, slot).wait()

        xb = xbuf[slot].astype(jnp.bfloat16)
        scores = jax.lax.dot_general(
            xb, keys16, (((1,), (1,)), ((), ())),
            preferred_element_type=jnp.float32) * _SCALE
        comp = jax.lax.dot_general(
            xb, mu16, (((1,), (1,)), ((), ())),
            preferred_element_type=jnp.float32)
        sbuf[slot] = scores
        s_copy(c, slot).start()
        obuf[slot] = comp * jnp.maximum(scores - gate, 0.0) + bias
        o_copy(c, slot).start()

        @pl.when(c + _NBUF < n_chunks)
        def _():
            in_copy(c + _NBUF, slot).start()

        return carry

    jax.lax.fori_loop(0, n_chunks, step, 0)

    for s in range(_NBUF):
        c = n_chunks - _NBUF + s
        slot = c % _NBUF
        s_copy(c, slot).wait()
        o_copy(c, slot).wait()


def kernel(x, mu_weight, sigma_weight, gate_param, mu_bias):
    x2 = x.reshape(-1, _IN)
    tokens = x2.shape[0]

    gate2 = gate_param.reshape(1, _OUT)
    bias2 = mu_bias.reshape(1, _OUT)

    scores, masked = pl.pallas_call(
        _body,
        in_specs=[
            pl.BlockSpec(memory_space=pl.ANY),
            pl.BlockSpec(memory_space=pltpu.MemorySpace.VMEM),
            pl.BlockSpec(memory_space=pltpu.MemorySpace.VMEM),
            pl.BlockSpec(memory_space=pltpu.MemorySpace.VMEM),
            pl.BlockSpec(memory_space=pltpu.MemorySpace.VMEM),
        ],
        out_specs=[
            pl.BlockSpec(memory_space=pl.ANY),
            pl.BlockSpec(memory_space=pl.ANY),
        ],
        out_shape=[
            jax.ShapeDtypeStruct((tokens, _OUT), jnp.float32),
            jax.ShapeDtypeStruct((tokens, _OUT), jnp.float32),
        ],
        scratch_shapes=[
            pltpu.VMEM((_NBUF, _CHUNK, _IN), jnp.float32),
            pltpu.VMEM((_NBUF, _CHUNK, _OUT), jnp.float32),
            pltpu.VMEM((_NBUF, _CHUNK, _OUT), jnp.float32),
            pltpu.SemaphoreType.DMA((_NBUF,)),
            pltpu.SemaphoreType.DMA((_NBUF,)),
            pltpu.SemaphoreType.DMA((_NBUF,)),
        ],
    )(x2, mu_weight, sigma_weight, gate2, bias2)

    final = masked.reshape(*x.shape[:-1], _OUT)
    return (final, scores, masked)


# speedup vs baseline: 1.0207x; 1.0207x over previous
import math

import jax
import jax.numpy as jnp
from jax.experimental import pallas as pl
from jax.experimental.pallas import tpu as pltpu

_IN = 512
_OUT = 512
_SCALE = 1.0 / math.sqrt(_IN)
_CHUNK = 1024
_NBUF = 8


def _body(x_hbm, mu_ref, sig_ref, gate_ref, bias_ref, scores_hbm, out_hbm,
          xbuf, sbuf, obuf, in_sems, ssems, osems):
    n_chunks = x_hbm.shape[0] // _CHUNK
    mu = mu_ref[:]
    keys16 = (mu * jax.nn.softplus(sig_ref[:])).astype(jnp.bfloat16)
    mu16 = mu.astype(jnp.bfloat16)
    gate = gate_ref[:]
    bias = bias_ref[:]

    def in_copy(c, slot):
        return pltpu.make_async_copy(
            x_hbm.at[pl.ds(c * _CHUNK, _CHUNK), :], xbuf.at[slot],
            in_sems.at[slot])

    def s_copy(c, slot):
        return pltpu.make_async_copy(
            sbuf.at[slot], scores_hbm.at[pl.ds(c * _CHUNK, _CHUNK), :],
            ssems.at[slot])

    def o_copy(c, slot):
        return pltpu.make_async_copy(
            obuf.at[slot], out_hbm.at[pl.ds(c * _CHUNK, _CHUNK), :],
            osems.at[slot])

    for s in range(_NBUF):
        in_copy(s, s).start()

    def step(c, carry):
        slot = jax.lax.rem(c, _NBUF)
        in_copy(c, slot).wait()

        @pl.when(c >= _NBUF)
        def _():
            s_copy(c, slot).wait()
            o_copy(c, slot).wait()

        xb = xbuf[slot].astype(jnp.bfloat16)
        scores = jax.lax.dot_general(
            xb, keys16, (((1,), (1,)), ((), ())),
            preferred_element_type=jnp.float32) * _SCALE
        comp = jax.lax.dot_general(
            xb, mu16, (((1,), (1,)), ((), ())),
            preferred_element_type=jnp.float32)
        sbuf[slot] = scores
        s_copy(c, slot).start()
        obuf[slot] = comp * jnp.maximum(scores - gate, 0.0) + bias
        o_copy(c, slot).start()

        @pl.when(c + _NBUF < n_chunks)
        def _():
            in_copy(c + _NBUF, slot).start()

        return carry

    jax.lax.fori_loop(0, n_chunks, step, 0)

    for s in range(_NBUF):
        c = n_chunks - _NBUF + s
        slot = c % _NBUF
        s_copy(c, slot).wait()
        o_copy(c, slot).wait()


def kernel(x, mu_weight, sigma_weight, gate_param, mu_bias):
    x2 = x.reshape(-1, _IN)
    tokens = x2.shape[0]

    gate2 = gate_param.reshape(1, _OUT)
    bias2 = mu_bias.reshape(1, _OUT)

    scores, masked = pl.pallas_call(
        _body,
        in_specs=[
            pl.BlockSpec(memory_space=pl.ANY),
            pl.BlockSpec(memory_space=pltpu.MemorySpace.VMEM),
            pl.BlockSpec(memory_space=pltpu.MemorySpace.VMEM),
            pl.BlockSpec(memory_space=pltpu.MemorySpace.VMEM),
            pl.BlockSpec(memory_space=pltpu.MemorySpace.VMEM),
        ],
        out_specs=[
            pl.BlockSpec(memory_space=pl.ANY),
            pl.BlockSpec(memory_space=pl.ANY),
        ],
        out_shape=[
            jax.ShapeDtypeStruct((tokens, _OUT), jnp.float32),
            jax.ShapeDtypeStruct((tokens, _OUT), jnp.float32),
        ],
        scratch_shapes=[
            pltpu.VMEM((_NBUF, _CHUNK, _IN), jnp.float32),
            pltpu.VMEM((_NBUF, _CHUNK, _OUT), jnp.float32),
            pltpu.VMEM((_NBUF, _CHUNK, _OUT), jnp.float32),
            pltpu.SemaphoreType.DMA((_NBUF,)),
            pltpu.SemaphoreType.DMA((_NBUF,)),
            pltpu.SemaphoreType.DMA((_NBUF,)),
        ],
    )(x2, mu_weight, sigma_weight, gate2, bias2)

    final = masked.reshape(*x.shape[:-1], _OUT)
    return (final, scores, masked)


# speedup vs baseline: 1.0216x; 1.0009x over previous
import math

import jax
import jax.numpy as jnp
from jax.experimental import pallas as pl
from jax.experimental.pallas import tpu as pltpu

_IN = 512
_OUT = 512
_SCALE = 1.0 / math.sqrt(_IN)
_CHUNK = 1024
_NBUF = 6


def _body(x_hbm, mu_ref, sig_ref, gate_ref, bias_ref, scores_hbm, out_hbm,
          xbuf, sbuf, obuf, in_sems, ssems, osems):
    n_chunks = x_hbm.shape[0] // _CHUNK
    mu = mu_ref[:]
    keys16 = (mu * jax.nn.softplus(sig_ref[:])).astype(jnp.bfloat16)
    mu16 = mu.astype(jnp.bfloat16)
    gate = gate_ref[:]
    bias = bias_ref[:]

    def in_copy(c, slot):
        return pltpu.make_async_copy(
            x_hbm.at[pl.ds(c * _CHUNK, _CHUNK), :], xbuf.at[slot],
            in_sems.at[slot])

    def s_copy(c, slot):
        return pltpu.make_async_copy(
            sbuf.at[slot], scores_hbm.at[pl.ds(c * _CHUNK, _CHUNK), :],
            ssems.at[slot])

    def o_copy(c, slot):
        return pltpu.make_async_copy(
            obuf.at[slot], out_hbm.at[pl.ds(c * _CHUNK, _CHUNK), :],
            osems.at[slot])

    for s in range(_NBUF):
        in_copy(s, s).start()

    def step(c, carry):
        slot = jax.lax.rem(c, _NBUF)
        in_copy(c, slot).wait()

        @pl.when(c >= _NBUF)
        def _():
            s_copy(c, slot).wait()
            o_copy(c, slot).wait()

        xb = xbuf[slot].astype(jnp.bfloat16)
        scores = jax.lax.dot_general(
            xb, keys16, (((1,), (1,)), ((), ())),
            preferred_element_type=jnp.float32) * _SCALE
        comp = jax.lax.dot_general(
            xb, mu16, (((1,), (1,)), ((), ())),
            preferred_element_type=jnp.float32)
        sbuf[slot] = scores
        s_copy(c, slot).start()
        obuf[slot] = comp * jnp.maximum(scores - gate, 0.0) + bias
        o_copy(c, slot).start()

        @pl.when(c + _NBUF < n_chunks)
        def _():
            in_copy(c + _NBUF, slot).start()

        return carry

    jax.lax.fori_loop(0, n_chunks, step, 0)

    for s in range(_NBUF):
        c = n_chunks - _NBUF + s
        slot = c % _NBUF
        s_copy(c, slot).wait()
        o_copy(c, slot).wait()


def kernel(x, mu_weight, sigma_weight, gate_param, mu_bias):
    x2 = x.reshape(-1, _IN)
    tokens = x2.shape[0]

    gate2 = gate_param.reshape(1, _OUT)
    bias2 = mu_bias.reshape(1, _OUT)

    scores, masked = pl.pallas_call(
        _body,
        in_specs=[
            pl.BlockSpec(memory_space=pl.ANY),
            pl.BlockSpec(memory_space=pltpu.MemorySpace.VMEM),
            pl.BlockSpec(memory_space=pltpu.MemorySpace.VMEM),
            pl.BlockSpec(memory_space=pltpu.MemorySpace.VMEM),
            pl.BlockSpec(memory_space=pltpu.MemorySpace.VMEM),
        ],
        out_specs=[
            pl.BlockSpec(memory_space=pl.ANY),
            pl.BlockSpec(memory_space=pl.ANY),
        ],
        out_shape=[
            jax.ShapeDtypeStruct((tokens, _OUT), jnp.float32),
            jax.ShapeDtypeStruct((tokens, _OUT), jnp.float32),
        ],
        scratch_shapes=[
            pltpu.VMEM((_NBUF, _CHUNK, _IN), jnp.float32),
            pltpu.VMEM((_NBUF, _CHUNK, _OUT), jnp.float32),
            pltpu.VMEM((_NBUF, _CHUNK, _OUT), jnp.float32),
            pltpu.SemaphoreType.DMA((_NBUF,)),
            pltpu.SemaphoreType.DMA((_NBUF,)),
            pltpu.SemaphoreType.DMA((_NBUF,)),
        ],
    )(x2, mu_weight, sigma_weight, gate2, bias2)

    final = masked.reshape(*x.shape[:-1], _OUT)
    return (final, scores, masked)


# speedup vs baseline: 1.0231x; 1.0015x over previous
import math

import jax
import jax.numpy as jnp
from jax.experimental import pallas as pl
from jax.experimental.pallas import tpu as pltpu

_IN = 512
_OUT = 512
_SCALE = 1.0 / math.sqrt(_IN)
_CHUNK = 1024
_NBUF = 6


def _body(x_hbm, mu_ref, sig_ref, gate_ref, bias_ref, scores_hbm, out_hbm,
          xbuf, sbuf, obuf, in_sems, ssems, osems):
    n_chunks = x_hbm.shape[0] // _CHUNK

    def in_copy(c, slot):
        return pltpu.make_async_copy(
            x_hbm.at[pl.ds(c * _CHUNK, _CHUNK), :], xbuf.at[slot],
            in_sems.at[slot])

    def s_copy(c, slot):
        return pltpu.make_async_copy(
            sbuf.at[slot], scores_hbm.at[pl.ds(c * _CHUNK, _CHUNK), :],
            ssems.at[slot])

    def o_copy(c, slot):
        return pltpu.make_async_copy(
            obuf.at[slot], out_hbm.at[pl.ds(c * _CHUNK, _CHUNK), :],
            osems.at[slot])

    for s in range(_NBUF):
        in_copy(s, s).start()

    mu = mu_ref[:]
    keys16 = (mu * jax.nn.softplus(sig_ref[:])).astype(jnp.bfloat16)
    mu16 = mu.astype(jnp.bfloat16)
    gate = gate_ref[:]
    bias = bias_ref[:]

    def step(c, carry):
        slot = jax.lax.rem(c, _NBUF)
        in_copy(c, slot).wait()

        @pl.when(c >= _NBUF)
        def _():
            s_copy(c, slot).wait()
            o_copy(c, slot).wait()

        xb = xbuf[slot].astype(jnp.bfloat16)
        scores = jax.lax.dot_general(
            xb, keys16, (((1,), (1,)), ((), ())),
            preferred_element_type=jnp.float32) * _SCALE
        comp = jax.lax.dot_general(
            xb, mu16, (((1,), (1,)), ((), ())),
            preferred_element_type=jnp.float32)
        sbuf[slot] = scores
        s_copy(c, slot).start()
        obuf[slot] = comp * jnp.maximum(scores - gate, 0.0) + bias
        o_copy(c, slot).start()

        @pl.when(c + _NBUF < n_chunks)
        def _():
            in_copy(c + _NBUF, slot).start()

        return carry

    jax.lax.fori_loop(0, n_chunks, step, 0)

    for s in range(_NBUF):
        c = n_chunks - _NBUF + s
        slot = c % _NBUF
        s_copy(c, slot).wait()
        o_copy(c, slot).wait()


def kernel(x, mu_weight, sigma_weight, gate_param, mu_bias):
    x2 = x.reshape(-1, _IN)
    tokens = x2.shape[0]

    gate2 = gate_param.reshape(1, _OUT)
    bias2 = mu_bias.reshape(1, _OUT)

    scores, masked = pl.pallas_call(
        _body,
        in_specs=[
            pl.BlockSpec(memory_space=pl.ANY),
            pl.BlockSpec(memory_space=pltpu.MemorySpace.VMEM),
            pl.BlockSpec(memory_space=pltpu.MemorySpace.VMEM),
            pl.BlockSpec(memory_space=pltpu.MemorySpace.VMEM),
            pl.BlockSpec(memory_space=pltpu.MemorySpace.VMEM),
        ],
        out_specs=[
            pl.BlockSpec(memory_space=pl.ANY),
            pl.BlockSpec(memory_space=pl.ANY),
        ],
        out_shape=[
            jax.ShapeDtypeStruct((tokens, _OUT), jnp.float32),
            jax.ShapeDtypeStruct((tokens, _OUT), jnp.float32),
        ],
        scratch_shapes=[
            pltpu.VMEM((_NBUF, _CHUNK, _IN), jnp.float32),
            pltpu.VMEM((_NBUF, _CHUNK, _OUT), jnp.float32),
            pltpu.VMEM((_NBUF, _CHUNK, _OUT), jnp.float32),
            pltpu.SemaphoreType.DMA((_NBUF,)),
            pltpu.SemaphoreType.DMA((_NBUF,)),
            pltpu.SemaphoreType.DMA((_NBUF,)),
        ],
    )(x2, mu_weight, sigma_weight, gate2, bias2)

    final = masked.reshape(*x.shape[:-1], _OUT)
    return (final, scores, masked)
